# baseline (speedup 1.0000x reference)
import functools

import jax
import jax.numpy as jnp
from jax import lax
from jax.experimental import pallas as pl
from jax.experimental.pallas import tpu as pltpu
from jax.experimental.pallas import tpu_sc as plsc

B = 64
S = 512
E = 128
N = B * S
NC = 2
NS = 16
NW = NC * NS
PER_W = N // NW
CHUNK = 128
NCHUNK = PER_W // CHUNK
NBUF = 7
PRE_AHEAD = 6
GAT_AHEAD = 4

_mesh = plsc.VectorSubcoreMesh(core_axis_name="c", subcore_axis_name="s")


SEQ_PER_W = PER_W // S
CH_PER_SEQ = S // CHUNK


@functools.partial(
    pl.kernel,
    out_type=jax.ShapeDtypeStruct((B, S, E), jnp.float32),
    mesh=_mesh,
    scratch_types=[
        pltpu.VMEM((SEQ_PER_W, S), jnp.int32),
        pltpu.VMEM_SHARED((S, E), jnp.float32),
        [pltpu.VMEM((CHUNK, E), jnp.float32) for _ in range(NBUF)],
        [pltpu.SemaphoreType.DMA for _ in range(NBUF)],
        [pltpu.SemaphoreType.DMA for _ in range(NBUF)],
        [pltpu.SemaphoreType.DMA for _ in range(NBUF)],
    ],
)
def _emb_lookup(x_hbm, tok_hbm, pos_hbm, out_hbm,
                idx_v, pos_sh, bufs, psems, gsems, ssems):
    sid = lax.axis_index("s")
    wid = sid * NC + lax.axis_index("c")
    b0 = wid * SEQ_PER_W

    idx_cp = pltpu.async_copy(x_hbm.at[pl.ds(b0, SEQ_PER_W)], idx_v,
                              gsems[NBUF - 1])

    prows = S // NS
    pltpu.sync_copy(pos_hbm.at[pl.ds(sid * prows, prows)],
                    pos_sh.at[pl.ds(sid * prows, prows)])

    plsc.subcore_barrier()

    rot = lax.rem(wid, CH_PER_SEQ)

    def chunk_coords(k):
        seq = k // CH_PER_SEQ
        po = lax.rem(jnp.int32(k) + rot, CH_PER_SEQ) * CHUNK
        return seq, po

    def start_prefill(k):
        b = k % NBUF
        seq, po = chunk_coords(k)
        return pltpu.async_copy(pos_sh.at[pl.ds(po, CHUNK)], bufs[b], psems[b])

    def start_gather_add(k):
        b = k % NBUF
        seq, po = chunk_coords(k)
        idx_row = idx_v.at[seq, pl.ds(po, CHUNK)]
        return pltpu.async_copy(tok_hbm.at[idx_row], bufs[b], gsems[b],
                                add=True)

    pre_cp = [None] * NBUF
    gather_cp = [None] * NBUF
    store_cp = [None] * NBUF

    for c in range(PRE_AHEAD):
        pre_cp[c % NBUF] = start_prefill(c)
    idx_cp.wait()
    for c in range(GAT_AHEAD):
        pre_cp[c % NBUF].wait()
        gather_cp[c % NBUF] = start_gather_add(c)

    for c in range(NCHUNK):
        b = c % NBUF
        if c + PRE_AHEAD < NCHUNK:
            bn = (c + PRE_AHEAD) % NBUF
            if store_cp[bn] is not None:
                store_cp[bn].wait()
            pre_cp[bn] = start_prefill(c + PRE_AHEAD)
        if c + GAT_AHEAD < NCHUNK:
            bm = (c + GAT_AHEAD) % NBUF
            pre_cp[bm].wait()
            gather_cp[bm] = start_gather_add(c + GAT_AHEAD)
        gather_cp[b].wait()
        seq, po = chunk_coords(c)
        store_cp[b] = pltpu.async_copy(
            bufs[b], out_hbm.at[b0 + seq, pl.ds(po, CHUNK)], ssems[b])

    for b in range(min(NBUF, NCHUNK)):
        store_cp[b].wait()


def kernel(x, token_table, pos_table):
    return _emb_lookup(x.astype(jnp.int32), token_table, pos_table)


# speedup vs baseline: 1.0009x; 1.0009x over previous
import functools

import jax
import jax.numpy as jnp
from jax import lax
from jax.experimental import pallas as pl
from jax.experimental.pallas import tpu as pltpu
from jax.experimental.pallas import tpu_sc as plsc

B = 64
S = 512
E = 128
N = B * S
NC = 2
NS = 16
NW = NC * NS
PER_W = N // NW
CHUNK = 128
NCHUNK = PER_W // CHUNK
NBUF = 7
PRE_AHEAD = 5
GAT_AHEAD = 3

_mesh = plsc.VectorSubcoreMesh(core_axis_name="c", subcore_axis_name="s")


SEQ_PER_W = PER_W // S
CH_PER_SEQ = S // CHUNK


@functools.partial(
    pl.kernel,
    out_type=jax.ShapeDtypeStruct((B, S, E), jnp.float32),
    mesh=_mesh,
    scratch_types=[
        pltpu.VMEM((SEQ_PER_W, S), jnp.int32),
        pltpu.VMEM_SHARED((S, E), jnp.float32),
        [pltpu.VMEM((CHUNK, E), jnp.float32) for _ in range(NBUF)],
        [pltpu.SemaphoreType.DMA for _ in range(NBUF)],
        [pltpu.SemaphoreType.DMA for _ in range(NBUF)],
        [pltpu.SemaphoreType.DMA for _ in range(NBUF)],
    ],
)
def _emb_lookup(x_hbm, tok_hbm, pos_hbm, out_hbm,
                idx_v, pos_sh, bufs, psems, gsems, ssems):
    sid = lax.axis_index("s")
    wid = sid * NC + lax.axis_index("c")
    b0 = wid * SEQ_PER_W

    idx_cp = pltpu.async_copy(x_hbm.at[pl.ds(b0, SEQ_PER_W)], idx_v,
                              gsems[NBUF - 1])

    prows = S // NS
    pltpu.sync_copy(pos_hbm.at[pl.ds(sid * prows, prows)],
                    pos_sh.at[pl.ds(sid * prows, prows)])

    plsc.subcore_barrier()

    rot = lax.rem(wid, CH_PER_SEQ)

    def chunk_coords(k):
        seq = k // CH_PER_SEQ
        po = lax.rem(jnp.int32(k) + rot, CH_PER_SEQ) * CHUNK
        return seq, po

    def start_prefill(k):
        b = k % NBUF
        seq, po = chunk_coords(k)
        return pltpu.async_copy(pos_sh.at[pl.ds(po, CHUNK)], bufs[b], psems[b])

    def start_gather_add(k):
        b = k % NBUF
        seq, po = chunk_coords(k)
        idx_row = idx_v.at[seq, pl.ds(po, CHUNK)]
        return pltpu.async_copy(tok_hbm.at[idx_row], bufs[b], gsems[b],
                                add=True)

    pre_cp = [None] * NBUF
    gather_cp = [None] * NBUF
    store_cp = [None] * NBUF

    for c in range(PRE_AHEAD):
        pre_cp[c % NBUF] = start_prefill(c)
    idx_cp.wait()
    for c in range(GAT_AHEAD):
        pre_cp[c % NBUF].wait()
        gather_cp[c % NBUF] = start_gather_add(c)

    for c in range(NCHUNK):
        b = c % NBUF
        if c + PRE_AHEAD < NCHUNK:
            bn = (c + PRE_AHEAD) % NBUF
            if store_cp[bn] is not None:
                store_cp[bn].wait()
            pre_cp[bn] = start_prefill(c + PRE_AHEAD)
        if c + GAT_AHEAD < NCHUNK:
            bm = (c + GAT_AHEAD) % NBUF
            pre_cp[bm].wait()
            gather_cp[bm] = start_gather_add(c + GAT_AHEAD)
        gather_cp[b].wait()
        seq, po = chunk_coords(c)
        store_cp[b] = pltpu.async_copy(
            bufs[b], out_hbm.at[b0 + seq, pl.ds(po, CHUNK)], ssems[b])

    for b in range(min(NBUF, NCHUNK)):
        store_cp[b].wait()


def kernel(x, token_table, pos_table):
    return _emb_lookup(x.astype(jnp.int32), token_table, pos_table)
